# baseline (speedup 1.0000x reference)
import functools

import jax
import jax.numpy as jnp
from jax.experimental import pallas as pl
from jax.experimental.pallas import tpu as pltpu

_CDT = jnp.bfloat16


def _mm(a, b):
    return jax.lax.dot_general(
        a.astype(_CDT), b.astype(_CDT),
        ((( 1,), (0,)), ((), ())),
        preferred_element_type=jnp.float32)


def _stats_body(x_ref, ea_ref, sx_ref, se_ref):
    g = pl.program_id(0)
    xb = x_ref[0]
    eb = ea_ref[0]
    px1 = jnp.sum(xb, axis=0, keepdims=True)
    px2 = jnp.sum(xb * xb, axis=0, keepdims=True)
    pe1 = jnp.sum(eb, axis=0, keepdims=True)
    pe2 = jnp.sum(eb * eb, axis=0, keepdims=True)

    @pl.when(g == 0)
    def _init():
        sx_ref[...] = jnp.zeros_like(sx_ref)
        se_ref[...] = jnp.zeros_like(se_ref)

    sx_ref[0:1, :] += px1
    sx_ref[1:2, :] += px2
    se_ref[0:1, :] += pe1
    se_ref[1:2, :] += pe2


def _bn_scale_shift(s1, s2, n, gamma, beta):
    mu = s1 / n
    var = s2 / n - mu * mu
    sc = gamma * jax.lax.rsqrt(var + 1e-5)
    return sc, beta - mu * sc


def _main_body(nw, names, npg, mpg,
               x_ref, ea_ref, rowT_ref, colT_ref, colH_ref,
               selr_ref, selc_ref, selp_ref, bnx_ref, bne_ref,
               *rest):
    w = dict(zip(names, rest[:nw]))
    out_ref = rest[nw]

    def W(name):
        return w[name][...]

    x = x_ref[0] * bnx_ref[0:1, :] + bnx_ref[1:2, :]
    e = (ea_ref[0] * bne_ref[0:1, :] + bne_ref[1:2, :]).astype(_CDT)

    rowT = rowT_ref[0]
    colT = colT_ref[0]
    colH = colH_ref[0]
    iota_en = jax.lax.broadcasted_iota(jnp.int32, (mpg, npg), 1)
    p_row = (iota_en == rowT).astype(_CDT)
    p_col = (iota_en == colT).astype(_CDT)
    mask_s = jax.lax.broadcasted_iota(jnp.int32, (npg, mpg), 0) == colH
    p_scat = mask_s.astype(_CDT)
    cnt = jnp.sum(mask_s.astype(jnp.float32), axis=1, keepdims=True)
    inv_cnt = 1.0 / jnp.maximum(cnt, 1.0)

    for i in range(1, 6):
        xr = _mm(p_row, x).astype(_CDT)
        xc = _mm(p_col, x).astype(_CDT)
        t = (_mm(xr, W(f'e{i}_wr')) + _mm(xc, W(f'e{i}_wc'))
             + _mm(e, W(f'e{i}_we')) + W(f'e{i}_b1'))
        t = jnp.maximum(t, 0.0)
        t = jnp.maximum(_mm(t, W(f'e{i}_w2')) + W(f'e{i}_b2'), 0.0)
        e = (_mm(t, W(f'e{i}_w3')) + W(f'e{i}_b3')).astype(_CDT)
        h = jnp.maximum(_mm(xr, W(f'n{i}_mx')) + _mm(e, W(f'n{i}_me'))
                        + W(f'n{i}_c1'), 0.0)
        h = jnp.maximum(_mm(h, W(f'n{i}_m2')) + W(f'n{i}_c2'), 0.0)
        h = _mm(h, W(f'n{i}_m3')) + W(f'n{i}_c3')
        agg = _mm(p_scat, h) * inv_cnt
        z = jnp.maximum(_mm(x, W(f'n{i}_nx')) + _mm(agg, W(f'n{i}_na'))
                        + W(f'n{i}_d1'), 0.0)
        x = _mm(z, W(f'n{i}_n2')) + W(f'n{i}_d2')

    selr = selr_ref[0]
    selc = selc_ref[0]
    selp = selp_ref[0]
    iota_sn = jax.lax.broadcasted_iota(jnp.int32, (8, npg), 1)
    iota_se = jax.lax.broadcasted_iota(jnp.int32, (8, mpg), 1)
    xr6 = _mm((iota_sn == selr).astype(_CDT), x).astype(_CDT)
    xc6 = _mm((iota_sn == selc).astype(_CDT), x).astype(_CDT)
    e6 = _mm((iota_se == selp).astype(_CDT), e).astype(_CDT)
    t = (_mm(xr6, W('e6_wr')) + _mm(xc6, W('e6_wc'))
         + _mm(e6, W('e6_we')) + W('e6_b1'))
    t = jnp.maximum(t, 0.0)
    t = jnp.maximum(_mm(t, W('e6_w2')) + W('e6_b2'), 0.0)
    t = _mm(t, W('e6_w3')) + W('e6_b3')
    pooled = jnp.sum(t[0:2, :], axis=0, keepdims=True)
    y = jnp.maximum(_mm(pooled, W('l1_w')) + W('l1_b'), 0.0)
    y = _mm(y, W('l2_w')) + W('l2_b')
    out_ref[0] = y


def kernel(x, edge_index, edge_attr, g, e_target, edge_type, batch, params):
    B = g.shape[0]
    N, DN = x.shape
    E, DE = edge_attr.shape
    NPG = N // B
    MPG = E // B

    row = edge_index[0].astype(jnp.int32).reshape(B, MPG)
    col = edge_index[1].astype(jnp.int32).reshape(B, MPG)
    node_off = (jnp.arange(B, dtype=jnp.int32) * NPG)[:, None]
    row_l = row - node_off
    col_l = col - node_off
    sel_idx = jnp.nonzero(edge_type == 0, size=2 * B)[0].astype(jnp.int32)
    pos_l = sel_idx.reshape(B, 2) % MPG
    rsel = jnp.take_along_axis(row_l, pos_l, axis=1)
    csel = jnp.take_along_axis(col_l, pos_l, axis=1)
    pad = jnp.zeros((B, 6), jnp.int32)
    selr = jnp.concatenate([rsel, pad], axis=1).reshape(B, 8, 1)
    selc = jnp.concatenate([csel, pad], axis=1).reshape(B, 8, 1)
    selp = jnp.concatenate([pos_l, pad], axis=1).reshape(B, 8, 1)

    x3 = x.reshape(B, NPG, DN)
    ea3 = edge_attr.reshape(B, MPG, DE)

    sum_x, sum_e = pl.pallas_call(
        _stats_body,
        grid=(B,),
        in_specs=[
            pl.BlockSpec((1, NPG, DN), lambda i: (i, 0, 0)),
            pl.BlockSpec((1, MPG, DE), lambda i: (i, 0, 0)),
        ],
        out_specs=[
            pl.BlockSpec((8, DN), lambda i: (0, 0)),
            pl.BlockSpec((8, DE), lambda i: (0, 0)),
        ],
        out_shape=[
            jax.ShapeDtypeStruct((8, DN), jnp.float32),
            jax.ShapeDtypeStruct((8, DE), jnp.float32),
        ],
    )(x3, ea3)
    sx, bx = _bn_scale_shift(sum_x[0], sum_x[1], float(N),
                             params['bn_node'][0], params['bn_node'][1])
    se, be = _bn_scale_shift(sum_e[0], sum_e[1], float(E),
                             params['bn_edge'][0], params['bn_edge'][1])
    bnx = jnp.concatenate([sx[None, :], bx[None, :],
                           jnp.zeros((6, DN), jnp.float32)], axis=0)
    bne = jnp.concatenate([se[None, :], be[None, :],
                           jnp.zeros((6, DE), jnp.float32)], axis=0)

    names, arrays = [], []

    def add_w(name, arr):
        names.append(name)
        arrays.append(arr.T.astype(_CDT))

    def add_b(name, arr):
        names.append(name)
        arrays.append(arr.reshape(1, -1).astype(jnp.float32))

    for i in range(1, 7):
        p = params[f'meta{i}']
        (W1, b1), (W2, b2), (W3, b3) = p['edge']
        add_w(f'e{i}_wr', W1[:, :DN])
        add_w(f'e{i}_wc', W1[:, DN:2 * DN])
        add_w(f'e{i}_we', W1[:, 2 * DN:])
        add_b(f'e{i}_b1', b1)
        add_w(f'e{i}_w2', W2)
        add_b(f'e{i}_b2', b2)
        add_w(f'e{i}_w3', W3)
        add_b(f'e{i}_b3', b3)
        if 'node' in p:
            (M1, c1), (M2, c2), (M3, c3) = p['node']['m1']
            add_w(f'n{i}_mx', M1[:, :DN])
            add_w(f'n{i}_me', M1[:, DN:])
            add_b(f'n{i}_c1', c1)
            add_w(f'n{i}_m2', M2)
            add_b(f'n{i}_c2', c2)
            add_w(f'n{i}_m3', M3)
            add_b(f'n{i}_c3', c3)
            (N1, d1), (N2, d2) = p['node']['m2']
            add_w(f'n{i}_nx', N1[:, :DN])
            add_w(f'n{i}_na', N1[:, DN:])
            add_b(f'n{i}_d1', d1)
            add_w(f'n{i}_n2', N2)
            add_b(f'n{i}_d2', d2)
    add_w('l1_w', params['lin1'][0])
    add_b('l1_b', params['lin1'][1])
    add_w('l2_w', params['lin2'][0])
    add_b('l2_b', params['lin2'][1])
    nw = len(names)

    const = lambda shape: pl.BlockSpec(shape, lambda i: tuple(0 for _ in shape))
    in_specs = [
        pl.BlockSpec((1, NPG, DN), lambda i: (i, 0, 0)),
        pl.BlockSpec((1, MPG, DE), lambda i: (i, 0, 0)),
        pl.BlockSpec((1, MPG, 1), lambda i: (i, 0, 0)),
        pl.BlockSpec((1, MPG, 1), lambda i: (i, 0, 0)),
        pl.BlockSpec((1, 1, MPG), lambda i: (i, 0, 0)),
        pl.BlockSpec((1, 8, 1), lambda i: (i, 0, 0)),
        pl.BlockSpec((1, 8, 1), lambda i: (i, 0, 0)),
        pl.BlockSpec((1, 8, 1), lambda i: (i, 0, 0)),
        const((8, DN)),
        const((8, DE)),
    ] + [const(a.shape) for a in arrays]

    y3 = pl.pallas_call(
        functools.partial(_main_body, nw, tuple(names), NPG, MPG),
        grid=(B,),
        in_specs=in_specs,
        out_specs=pl.BlockSpec((1, 1, 1), lambda i: (i, 0, 0)),
        out_shape=jax.ShapeDtypeStruct((B, 1, 1), jnp.float32),
        compiler_params=pltpu.CompilerParams(
            dimension_semantics=("parallel",)),
    )(x3, ea3,
      row_l.reshape(B, MPG, 1), col_l.reshape(B, MPG, 1),
      col_l.reshape(B, 1, MPG),
      selr, selc, selp, bnx, bne, *arrays)
    return y3.reshape(B, 1)


# speedup vs baseline: 1.3563x; 1.3563x over previous
import functools

import jax
import jax.numpy as jnp
from jax.experimental import pallas as pl
from jax.experimental.pallas import tpu as pltpu

_CDT = jnp.bfloat16


def _mm(a, b):
    return jax.lax.dot_general(
        a.astype(_CDT), b.astype(_CDT),
        ((( 1,), (0,)), ((), ())),
        preferred_element_type=jnp.float32)


def _stats_body(x_ref, ea_ref, sx_ref, se_ref):
    g = pl.program_id(0)
    xb = x_ref[0]
    eb = ea_ref[0]
    px1 = jnp.sum(xb, axis=0, keepdims=True)
    px2 = jnp.sum(xb * xb, axis=0, keepdims=True)
    pe1 = jnp.sum(eb, axis=0, keepdims=True)
    pe2 = jnp.sum(eb * eb, axis=0, keepdims=True)

    @pl.when(g == 0)
    def _init():
        sx_ref[...] = jnp.zeros_like(sx_ref)
        se_ref[...] = jnp.zeros_like(se_ref)

    sx_ref[0:1, :] += px1
    sx_ref[1:2, :] += px2
    se_ref[0:1, :] += pe1
    se_ref[1:2, :] += pe2


def _bn_scale_shift(s1, s2, n, gamma, beta):
    mu = s1 / n
    var = s2 / n - mu * mu
    sc = gamma * jax.lax.rsqrt(var + 1e-5)
    return sc, beta - mu * sc


def _main_body(nw, names, npg, mpg,
               x_ref, ea_ref, rowT_ref, colT_ref, colH_ref,
               selr_ref, selc_ref, selp_ref, bnx_ref, bne_ref,
               *rest):
    w = dict(zip(names, rest[:nw]))
    out_ref = rest[nw]

    def W(name):
        return w[name][...]

    x = x_ref[0] * bnx_ref[0:1, :] + bnx_ref[1:2, :]
    e2 = (ea_ref[0] * bne_ref[0:1, :] + bne_ref[1:2, :]).astype(_CDT)

    rowT = rowT_ref[0]
    colT = colT_ref[0]
    colH = colH_ref[0]
    iota_en = jax.lax.broadcasted_iota(jnp.int32, (mpg, npg), 1)
    p_row = (iota_en == rowT).astype(_CDT)
    p_col = (iota_en == colT).astype(_CDT)
    mask_s = jax.lax.broadcasted_iota(jnp.int32, (npg, mpg), 0) == colH
    p_scat = mask_s.astype(_CDT)
    cnt = jnp.sum(mask_s.astype(jnp.float32), axis=1, keepdims=True)
    inv_cnt = 1.0 / jnp.maximum(cnt, 1.0)
    nz = cnt * inv_cnt

    for i in range(1, 6):
        xb = x.astype(_CDT)
        xwr = _mm(xb, W(f'e{i}_wr')).astype(_CDT)
        xwc = _mm(xb, W(f'e{i}_wc')).astype(_CDT)
        xmx = _mm(xb, W(f'n{i}_mx')).astype(_CDT)
        t = (_mm(p_row, xwr) + _mm(p_col, xwc)
             + _mm(e2, W(f'e{i}_we')) + W(f'e{i}_b1'))
        t = jnp.maximum(t, 0.0)
        e2n = jnp.maximum(_mm(t, W(f'e{i}_w2')) + W(f'e{i}_b2'),
                          0.0).astype(_CDT)
        h = jnp.maximum(_mm(p_row, xmx) + _mm(e2n, W(f'n{i}_w3me'))
                        + W(f'n{i}_c1'), 0.0)
        h = jnp.maximum(_mm(h, W(f'n{i}_m2')) + W(f'n{i}_c2'), 0.0)
        agg = _mm(p_scat, h) * inv_cnt
        z = jnp.maximum(_mm(xb, W(f'n{i}_nx')) + _mm(agg, W(f'n{i}_m3na'))
                        + nz * W(f'n{i}_c3na') + W(f'n{i}_d1'), 0.0)
        x = _mm(z, W(f'n{i}_n2')) + W(f'n{i}_d2')
        e2 = e2n

    selr = selr_ref[0]
    selc = selc_ref[0]
    selp = selp_ref[0]
    iota_sn = jax.lax.broadcasted_iota(jnp.int32, (8, npg), 1)
    iota_se = jax.lax.broadcasted_iota(jnp.int32, (8, mpg), 1)
    xb = x.astype(_CDT)
    xr6 = _mm((iota_sn == selr).astype(_CDT), xb).astype(_CDT)
    xc6 = _mm((iota_sn == selc).astype(_CDT), xb).astype(_CDT)
    e2s = _mm((iota_se == selp).astype(_CDT), e2).astype(_CDT)
    t = (_mm(xr6, W('e6_wr')) + _mm(xc6, W('e6_wc'))
         + _mm(e2s, W('e6_we')) + W('e6_b1'))
    t = jnp.maximum(t, 0.0)
    t = jnp.maximum(_mm(t, W('e6_w2')) + W('e6_b2'), 0.0)
    t = _mm(t, W('e6_w3')) + W('e6_b3')
    pooled = jnp.sum(t[0:2, :], axis=0, keepdims=True)
    y = jnp.maximum(_mm(pooled, W('l1_w')) + W('l1_b'), 0.0)
    y = _mm(y, W('l2_w')) + W('l2_b')
    out_ref[0] = y


def kernel(x, edge_index, edge_attr, g, e_target, edge_type, batch, params):
    B = g.shape[0]
    N, DN = x.shape
    E, DE = edge_attr.shape
    NPG = N // B
    MPG = E // B

    row = edge_index[0].astype(jnp.int32).reshape(B, MPG)
    col = edge_index[1].astype(jnp.int32).reshape(B, MPG)
    node_off = (jnp.arange(B, dtype=jnp.int32) * NPG)[:, None]
    row_l = row - node_off
    col_l = col - node_off
    sel_idx = jnp.nonzero(edge_type == 0, size=2 * B)[0].astype(jnp.int32)
    pos_l = sel_idx.reshape(B, 2) % MPG
    rsel = jnp.take_along_axis(row_l, pos_l, axis=1)
    csel = jnp.take_along_axis(col_l, pos_l, axis=1)
    pad = jnp.zeros((B, 6), jnp.int32)
    selr = jnp.concatenate([rsel, pad], axis=1).reshape(B, 8, 1)
    selc = jnp.concatenate([csel, pad], axis=1).reshape(B, 8, 1)
    selp = jnp.concatenate([pos_l, pad], axis=1).reshape(B, 8, 1)

    x3 = x.reshape(B, NPG, DN)
    ea3 = edge_attr.reshape(B, MPG, DE)

    sum_x, sum_e = pl.pallas_call(
        _stats_body,
        grid=(B,),
        in_specs=[
            pl.BlockSpec((1, NPG, DN), lambda i: (i, 0, 0)),
            pl.BlockSpec((1, MPG, DE), lambda i: (i, 0, 0)),
        ],
        out_specs=[
            pl.BlockSpec((8, DN), lambda i: (0, 0)),
            pl.BlockSpec((8, DE), lambda i: (0, 0)),
        ],
        out_shape=[
            jax.ShapeDtypeStruct((8, DN), jnp.float32),
            jax.ShapeDtypeStruct((8, DE), jnp.float32),
        ],
    )(x3, ea3)
    sx, bx = _bn_scale_shift(sum_x[0], sum_x[1], float(N),
                             params['bn_node'][0], params['bn_node'][1])
    se, be = _bn_scale_shift(sum_e[0], sum_e[1], float(E),
                             params['bn_edge'][0], params['bn_edge'][1])
    bnx = jnp.concatenate([sx[None, :], bx[None, :],
                           jnp.zeros((6, DN), jnp.float32)], axis=0)
    bne = jnp.concatenate([se[None, :], be[None, :],
                           jnp.zeros((6, DE), jnp.float32)], axis=0)

    names, arrays = [], []

    def add_w(name, arr):
        names.append(name)
        arrays.append(arr.astype(_CDT))

    def add_b(name, arr):
        names.append(name)
        arrays.append(arr.reshape(1, -1).astype(jnp.float32))

    w3_prev = None
    b3_prev = None
    for i in range(1, 7):
        p = params[f'meta{i}']
        (W1, b1), (W2, b2), (W3, b3) = p['edge']
        add_w(f'e{i}_wr', W1[:, :DN].T)
        add_w(f'e{i}_wc', W1[:, DN:2 * DN].T)
        we = W1[:, 2 * DN:].T
        if w3_prev is None:
            add_w(f'e{i}_we', we)
            add_b(f'e{i}_b1', b1)
        else:
            add_w(f'e{i}_we', w3_prev @ we)
            add_b(f'e{i}_b1', b1 + b3_prev @ we)
        add_w(f'e{i}_w2', W2.T)
        add_b(f'e{i}_b2', b2)
        w3t = W3.T
        if 'node' in p:
            (M1, c1), (M2, c2), (M3, c3) = p['node']['m1']
            me = M1[:, DN:].T
            add_w(f'n{i}_mx', M1[:, :DN].T)
            add_w(f'n{i}_w3me', w3t @ me)
            add_b(f'n{i}_c1', c1 + b3 @ me)
            add_w(f'n{i}_m2', M2.T)
            add_b(f'n{i}_c2', c2)
            (N1, d1), (N2, d2) = p['node']['m2']
            na = N1[:, DN:].T
            add_w(f'n{i}_m3na', M3.T @ na)
            add_b(f'n{i}_c3na', c3 @ na)
            add_w(f'n{i}_nx', N1[:, :DN].T)
            add_b(f'n{i}_d1', d1)
            add_w(f'n{i}_n2', N2.T)
            add_b(f'n{i}_d2', d2)
            w3_prev, b3_prev = w3t, b3
        else:
            add_w(f'e{i}_w3', w3t)
            add_b(f'e{i}_b3', b3)
    add_w('l1_w', params['lin1'][0].T)
    add_b('l1_b', params['lin1'][1])
    add_w('l2_w', params['lin2'][0].T)
    add_b('l2_b', params['lin2'][1])
    nw = len(names)

    const = lambda shape: pl.BlockSpec(shape, lambda i: tuple(0 for _ in shape))
    in_specs = [
        pl.BlockSpec((1, NPG, DN), lambda i: (i, 0, 0)),
        pl.BlockSpec((1, MPG, DE), lambda i: (i, 0, 0)),
        pl.BlockSpec((1, MPG, 1), lambda i: (i, 0, 0)),
        pl.BlockSpec((1, MPG, 1), lambda i: (i, 0, 0)),
        pl.BlockSpec((1, 1, MPG), lambda i: (i, 0, 0)),
        pl.BlockSpec((1, 8, 1), lambda i: (i, 0, 0)),
        pl.BlockSpec((1, 8, 1), lambda i: (i, 0, 0)),
        pl.BlockSpec((1, 8, 1), lambda i: (i, 0, 0)),
        const((8, DN)),
        const((8, DE)),
    ] + [const(a.shape) for a in arrays]

    y3 = pl.pallas_call(
        functools.partial(_main_body, nw, tuple(names), NPG, MPG),
        grid=(B,),
        in_specs=in_specs,
        out_specs=pl.BlockSpec((1, 1, 1), lambda i: (i, 0, 0)),
        out_shape=jax.ShapeDtypeStruct((B, 1, 1), jnp.float32),
        compiler_params=pltpu.CompilerParams(
            dimension_semantics=("parallel",)),
    )(x3, ea3,
      row_l.reshape(B, MPG, 1), col_l.reshape(B, MPG, 1),
      col_l.reshape(B, 1, MPG),
      selr, selc, selp, bnx, bne, *arrays)
    return y3.reshape(B, 1)


# speedup vs baseline: 1.9138x; 1.4111x over previous
import functools

import jax
import jax.numpy as jnp
import numpy as np
from jax.experimental import pallas as pl
from jax.experimental.pallas import tpu as pltpu
from jax.sharding import Mesh, PartitionSpec as P

_CDT = jnp.bfloat16


def _mm(a, b):
    return jax.lax.dot_general(
        a.astype(_CDT), b.astype(_CDT),
        ((( 1,), (0,)), ((), ())),
        preferred_element_type=jnp.float32)


def _stats_body(x_ref, ea_ref, sx_ref, se_ref):
    g = pl.program_id(0)
    xb = x_ref[0]
    eb = ea_ref[0]
    px1 = jnp.sum(xb, axis=0, keepdims=True)
    px2 = jnp.sum(xb * xb, axis=0, keepdims=True)
    pe1 = jnp.sum(eb, axis=0, keepdims=True)
    pe2 = jnp.sum(eb * eb, axis=0, keepdims=True)

    @pl.when(g == 0)
    def _init():
        sx_ref[...] = jnp.zeros_like(sx_ref)
        se_ref[...] = jnp.zeros_like(se_ref)

    sx_ref[0:1, :] += px1
    sx_ref[1:2, :] += px2
    se_ref[0:1, :] += pe1
    se_ref[1:2, :] += pe2


def _bn_scale_shift(s1, s2, n, gamma, beta):
    mu = s1 / n
    var = s2 / n - mu * mu
    sc = gamma * jax.lax.rsqrt(var + 1e-5)
    return sc, beta - mu * sc


def _main_body(nw, names, npg, mpg,
               x_ref, ea_ref, rowT_ref, colT_ref, colH_ref,
               selr_ref, selc_ref, selp_ref, bnx_ref, bne_ref,
               *rest):
    w = dict(zip(names, rest[:nw]))
    out_ref = rest[nw]

    def W(name):
        return w[name][...]

    x = x_ref[0] * bnx_ref[0:1, :] + bnx_ref[1:2, :]
    e2 = (ea_ref[0] * bne_ref[0:1, :] + bne_ref[1:2, :]).astype(_CDT)

    rowT = rowT_ref[0]
    colT = colT_ref[0]
    colH = colH_ref[0]
    iota_en = jax.lax.broadcasted_iota(jnp.int32, (mpg, npg), 1)
    p_row = (iota_en == rowT).astype(_CDT)
    p_col = (iota_en == colT).astype(_CDT)
    mask_s = jax.lax.broadcasted_iota(jnp.int32, (npg, mpg), 0) == colH
    p_scat = mask_s.astype(_CDT)
    cnt = jnp.sum(mask_s.astype(jnp.float32), axis=1, keepdims=True)
    inv_cnt = 1.0 / jnp.maximum(cnt, 1.0)
    nz = cnt * inv_cnt

    for i in range(1, 6):
        xb = x.astype(_CDT)
        xwr = _mm(xb, W(f'e{i}_wr')).astype(_CDT)
        xwc = _mm(xb, W(f'e{i}_wc')).astype(_CDT)
        xmx = _mm(xb, W(f'n{i}_mx')).astype(_CDT)
        t = (_mm(p_row, xwr) + _mm(p_col, xwc)
             + _mm(e2, W(f'e{i}_we')) + W(f'e{i}_b1'))
        t = jnp.maximum(t, 0.0)
        e2n = jnp.maximum(_mm(t, W(f'e{i}_w2')) + W(f'e{i}_b2'),
                          0.0).astype(_CDT)
        h = jnp.maximum(_mm(p_row, xmx) + _mm(e2n, W(f'n{i}_w3me'))
                        + W(f'n{i}_c1'), 0.0)
        h = jnp.maximum(_mm(h, W(f'n{i}_m2')) + W(f'n{i}_c2'), 0.0)
        agg = _mm(p_scat, h) * inv_cnt
        z = jnp.maximum(_mm(xb, W(f'n{i}_nx')) + _mm(agg, W(f'n{i}_m3na'))
                        + nz * W(f'n{i}_c3na') + W(f'n{i}_d1'), 0.0)
        x = _mm(z, W(f'n{i}_n2')) + W(f'n{i}_d2')
        e2 = e2n

    selr = selr_ref[0]
    selc = selc_ref[0]
    selp = selp_ref[0]
    iota_sn = jax.lax.broadcasted_iota(jnp.int32, (8, npg), 1)
    iota_se = jax.lax.broadcasted_iota(jnp.int32, (8, mpg), 1)
    xb = x.astype(_CDT)
    xr6 = _mm((iota_sn == selr).astype(_CDT), xb).astype(_CDT)
    xc6 = _mm((iota_sn == selc).astype(_CDT), xb).astype(_CDT)
    e2s = _mm((iota_se == selp).astype(_CDT), e2).astype(_CDT)
    t = (_mm(xr6, W('e6_wr')) + _mm(xc6, W('e6_wc'))
         + _mm(e2s, W('e6_we')) + W('e6_b1'))
    t = jnp.maximum(t, 0.0)
    t = jnp.maximum(_mm(t, W('e6_w2')) + W('e6_b2'), 0.0)
    t = _mm(t, W('e6_w3')) + W('e6_b3')
    pooled = jnp.sum(t[0:2, :], axis=0, keepdims=True)
    y = jnp.maximum(_mm(pooled, W('l1_w')) + W('l1_b'), 0.0)
    y = _mm(y, W('l2_w')) + W('l2_b')
    out_ref[0] = y


def kernel(x, edge_index, edge_attr, g, e_target, edge_type, batch, params):
    B = g.shape[0]
    N, DN = x.shape
    E, DE = edge_attr.shape
    NPG = N // B
    MPG = E // B

    row = edge_index[0].astype(jnp.int32).reshape(B, MPG)
    col = edge_index[1].astype(jnp.int32).reshape(B, MPG)
    node_off = (jnp.arange(B, dtype=jnp.int32) * NPG)[:, None]
    row_l = row - node_off
    col_l = col - node_off
    sel_idx = jnp.nonzero(edge_type == 0, size=2 * B)[0].astype(jnp.int32)
    pos_l = sel_idx.reshape(B, 2) % MPG
    rsel = jnp.take_along_axis(row_l, pos_l, axis=1)
    csel = jnp.take_along_axis(col_l, pos_l, axis=1)
    pad = jnp.zeros((B, 6), jnp.int32)
    selr = jnp.concatenate([rsel, pad], axis=1).reshape(B, 8, 1)
    selc = jnp.concatenate([csel, pad], axis=1).reshape(B, 8, 1)
    selp = jnp.concatenate([pos_l, pad], axis=1).reshape(B, 8, 1)

    x3 = x.reshape(B, NPG, DN)
    ea3 = edge_attr.reshape(B, MPG, DE)

    names, arrays = [], []

    def add_w(name, arr):
        names.append(name)
        arrays.append(arr.astype(_CDT))

    def add_b(name, arr):
        names.append(name)
        arrays.append(arr.reshape(1, -1).astype(jnp.float32))

    w3_prev = None
    b3_prev = None
    for i in range(1, 7):
        p = params[f'meta{i}']
        (W1, b1), (W2, b2), (W3, b3) = p['edge']
        add_w(f'e{i}_wr', W1[:, :DN].T)
        add_w(f'e{i}_wc', W1[:, DN:2 * DN].T)
        we = W1[:, 2 * DN:].T
        if w3_prev is None:
            add_w(f'e{i}_we', we)
            add_b(f'e{i}_b1', b1)
        else:
            add_w(f'e{i}_we', w3_prev @ we)
            add_b(f'e{i}_b1', b1 + b3_prev @ we)
        add_w(f'e{i}_w2', W2.T)
        add_b(f'e{i}_b2', b2)
        w3t = W3.T
        if 'node' in p:
            (M1, c1), (M2, c2), (M3, c3) = p['node']['m1']
            me = M1[:, DN:].T
            add_w(f'n{i}_mx', M1[:, :DN].T)
            add_w(f'n{i}_w3me', w3t @ me)
            add_b(f'n{i}_c1', c1 + b3 @ me)
            add_w(f'n{i}_m2', M2.T)
            add_b(f'n{i}_c2', c2)
            (N1, d1), (N2, d2) = p['node']['m2']
            na = N1[:, DN:].T
            add_w(f'n{i}_m3na', M3.T @ na)
            add_b(f'n{i}_c3na', c3 @ na)
            add_w(f'n{i}_nx', N1[:, :DN].T)
            add_b(f'n{i}_d1', d1)
            add_w(f'n{i}_n2', N2.T)
            add_b(f'n{i}_d2', d2)
            w3_prev, b3_prev = w3t, b3
        else:
            add_w(f'e{i}_w3', w3t)
            add_b(f'e{i}_b3', b3)
    add_w('l1_w', params['lin1'][0].T)
    add_b('l1_b', params['lin1'][1])
    add_w('l2_w', params['lin2'][0].T)
    add_b('l2_b', params['lin2'][1])
    nw = len(names)

    const = lambda shape: pl.BlockSpec(shape, lambda i: tuple(0 for _ in shape))
    in_specs = [
        pl.BlockSpec((1, NPG, DN), lambda i: (i, 0, 0)),
        pl.BlockSpec((1, MPG, DE), lambda i: (i, 0, 0)),
        pl.BlockSpec((1, MPG, 1), lambda i: (i, 0, 0)),
        pl.BlockSpec((1, MPG, 1), lambda i: (i, 0, 0)),
        pl.BlockSpec((1, 1, MPG), lambda i: (i, 0, 0)),
        pl.BlockSpec((1, 8, 1), lambda i: (i, 0, 0)),
        pl.BlockSpec((1, 8, 1), lambda i: (i, 0, 0)),
        pl.BlockSpec((1, 8, 1), lambda i: (i, 0, 0)),
        const((8, DN)),
        const((8, DE)),
    ] + [const(a.shape) for a in arrays]

    devs = jax.devices()
    ndev = len(devs)
    while B % ndev != 0:
        ndev -= 1
    b_loc = B // ndev

    def call(x3l, ea3l, rowTl, colTl, colHl, selrl, selcl, selpl,
             gx, betax, ge, betae, *warrs):
        sum_x, sum_e = pl.pallas_call(
            _stats_body,
            grid=(b_loc,),
            in_specs=[
                pl.BlockSpec((1, NPG, DN), lambda i: (i, 0, 0)),
                pl.BlockSpec((1, MPG, DE), lambda i: (i, 0, 0)),
            ],
            out_specs=[
                pl.BlockSpec((8, DN), lambda i: (0, 0)),
                pl.BlockSpec((8, DE), lambda i: (0, 0)),
            ],
            out_shape=[
                jax.ShapeDtypeStruct((8, DN), jnp.float32),
                jax.ShapeDtypeStruct((8, DE), jnp.float32),
            ],
        )(x3l, ea3l)
        if ndev > 1:
            sum_x = jax.lax.psum(sum_x, 'd')
            sum_e = jax.lax.psum(sum_e, 'd')
        sx, bx = _bn_scale_shift(sum_x[0], sum_x[1], float(N), gx, betax)
        se, be = _bn_scale_shift(sum_e[0], sum_e[1], float(E), ge, betae)
        bnx = jnp.concatenate([sx[None, :], bx[None, :],
                               jnp.zeros((6, DN), jnp.float32)], axis=0)
        bne = jnp.concatenate([se[None, :], be[None, :],
                               jnp.zeros((6, DE), jnp.float32)], axis=0)
        return pl.pallas_call(
            functools.partial(_main_body, nw, tuple(names), NPG, MPG),
            grid=(b_loc,),
            in_specs=in_specs,
            out_specs=pl.BlockSpec((1, 1, 1), lambda i: (i, 0, 0)),
            out_shape=jax.ShapeDtypeStruct((b_loc, 1, 1), jnp.float32),
            compiler_params=pltpu.CompilerParams(
                dimension_semantics=("parallel",)),
        )(x3l, ea3l, rowTl, colTl, colHl, selrl, selcl, selpl,
          bnx, bne, *warrs)

    operands = (x3, ea3,
                row_l.reshape(B, MPG, 1), col_l.reshape(B, MPG, 1),
                col_l.reshape(B, 1, MPG), selr, selc, selp,
                params['bn_node'][0], params['bn_node'][1],
                params['bn_edge'][0], params['bn_edge'][1], *arrays)
    if ndev > 1:
        mesh = Mesh(np.array(devs[:ndev]), ('d',))
        sharded = (P('d'),) * 8 + (P(),) * (4 + len(arrays))
        call = jax.shard_map(call, mesh=mesh, in_specs=sharded,
                             out_specs=P('d'), check_vma=False)
    y3 = call(*operands)
    return y3.reshape(B, 1)
